# baseline (speedup 1.0000x reference)
import functools

import jax
import jax.numpy as jnp
from jax import lax
from jax.experimental import pallas as pl
from jax.experimental.pallas import tpu as pltpu
from jax.experimental.pallas import tpu_sc as plsc

BINS_ = 32
D_ = 128
B_ = 4
L_ = 512

NC = 2
NS = 16
LANES = 16
NW = NC * NS

ROWS = B_ * L_ * L_
PAIRS = B_ * L_
PAIRS_PER_W = PAIRS // NW
QUARTS = 4
QL = L_ // QUARTS
GROUP = 8


def _sc_body(ri_hbm, mask_hbm, table_hbm, out_hbm,
             ri_v, m_v, idx_v, rows_v, sem, osem):
    wid = lax.axis_index("s") * NC + lax.axis_index("c")
    b = wid // (L_ // PAIRS_PER_W)
    i0 = (wid % (L_ // PAIRS_PER_W)) * PAIRS_PER_W

    pltpu.sync_copy(ri_hbm.at[b], ri_v.at[pl.ds(0, L_)])
    pltpu.sync_copy(mask_hbm.at[b], m_v.at[pl.ds(0, L_)])

    def group_body(g, carry):
        gi = i0 + g * GROUP
        ri_g = ri_v[pl.ds(gi, LANES)]
        m_g = m_v[pl.ds(gi, LANES)]
        for p in range(GROUP):
            pair_work(gi + p, ri_g[p], m_g[p])
        return carry

    def pair_work(i, ri_i_s, m_i_s):
        ri_i = jnp.full((LANES,), ri_i_s, dtype=jnp.int32)
        m_i = jnp.full((LANES,), m_i_s, dtype=jnp.int32)

        for q in range(QUARTS):
            for jb in range(QL // LANES):
                sl = pl.ds(jb * LANES, LANES)
                ri_j = ri_v[pl.ds(q * QL + jb * LANES, LANES)]
                m_j = m_v[pl.ds(q * QL + jb * LANES, LANES)]
                d = jnp.clip(ri_j - ri_i, -BINS_, BINS_) + (BINS_ + 1)
                idx_v[q, sl] = d * (m_j * m_i)

        cps = [pltpu.make_async_copy(
                   table_hbm.at[idx_v.at[q]],
                   rows_v.at[pl.ds(q * QL, QL)], sem)
               for q in range(QUARTS)]
        for cp in cps:
            cp.start()
        for cp in cps:
            cp.wait()

        row0 = (b * L_ + i) * L_
        out_cp = pltpu.make_async_copy(rows_v, out_hbm.at[pl.ds(row0, L_)],
                                       osem)
        out_cp.start()
        out_cp.wait()

    lax.fori_loop(0, PAIRS_PER_W // GROUP, group_body, 0)


@jax.jit
def _rel_pos_sc(ri, mask_i32, table):
    mesh = plsc.VectorSubcoreMesh(core_axis_name="c", subcore_axis_name="s")
    f = pl.kernel(
        _sc_body,
        out_type=jax.ShapeDtypeStruct((ROWS, D_), jnp.float32),
        mesh=mesh,
        scratch_types=[
            pltpu.VMEM((L_ + LANES,), jnp.int32),
            pltpu.VMEM((L_ + LANES,), jnp.int32),
            pltpu.VMEM((QUARTS, QL), jnp.int32),
            pltpu.VMEM((L_, D_), jnp.float32),
            pltpu.SemaphoreType.DMA,
            pltpu.SemaphoreType.DMA,
        ],
    )
    return f(ri, mask_i32, table)


def kernel(residue_index, mask, table):
    out = _rel_pos_sc(residue_index, mask.astype(jnp.int32), table)
    return out.reshape(B_, L_, L_, D_)


# speedup vs baseline: 19.0902x; 19.0902x over previous
import jax
import jax.numpy as jnp
from jax import lax
from jax.experimental import pallas as pl
from jax.experimental.pallas import tpu as pltpu
from jax.experimental.pallas import tpu_sc as plsc

BINS_ = 32
D_ = 128
B_ = 4
L_ = 512

NC = 2
NS = 16
LANES = 16
NW = NC * NS

ROWS = B_ * L_ * L_
PAIRS = B_ * L_
PAIRS_PER_W = PAIRS // NW
HALVES = 2
HL = L_ // HALVES
GROUP = 2
VPR = D_ // LANES


def _sc_body(ri_hbm, mask_hbm, table_hbm, out_hbm,
             ri_v, m_v, table_v, rows_v, osem):
    wid = lax.axis_index("s") * NC + lax.axis_index("c")
    b = wid // (L_ // PAIRS_PER_W)
    i0 = (wid % (L_ // PAIRS_PER_W)) * PAIRS_PER_W

    pltpu.sync_copy(table_hbm, table_v)
    pltpu.sync_copy(ri_hbm.at[b], ri_v.at[pl.ds(0, L_)])
    pltpu.sync_copy(mask_hbm.at[b], m_v.at[pl.ds(0, L_)])

    def out_cp(i, h):
        row0 = (b * L_ + i) * L_ + h * HL
        return pltpu.make_async_copy(rows_v.at[h],
                                     out_hbm.at[pl.ds(row0, HL)], osem.at[h])

    def pair_work(i, ri_i_s, m_i_s):
        ri_i = jnp.full((LANES,), ri_i_s, dtype=jnp.int32)
        m_i = jnp.full((LANES,), m_i_s, dtype=jnp.int32)

        for h in range(HALVES):
            @pl.when(i > i0)
            def _wait(h=h):
                out_cp(i - 1, h).wait()

            def blk(jb, carry):
                jpos = h * HL + jb * LANES
                ri_j = ri_v[pl.ds(jpos, LANES)]
                m_j = m_v[pl.ds(jpos, LANES)]
                d = jnp.clip(ri_j - ri_i, -BINS_, BINS_) + (BINS_ + 1)
                idx16 = d * (m_j * m_i)
                r0 = jb * LANES
                for r in range(LANES):
                    s = idx16[r]
                    for c in range(VPR):
                        rows_v[h, r0 + r, pl.ds(c * LANES, LANES)] = (
                            table_v[s, pl.ds(c * LANES, LANES)])
                return carry

            lax.fori_loop(0, HL // LANES, blk, 0)
            out_cp(i, h).start()

    def group_body(g, carry):
        gi = i0 + g * GROUP
        ri_g = ri_v[pl.ds(gi, LANES)]
        m_g = m_v[pl.ds(gi, LANES)]
        for p in range(GROUP):
            pair_work(gi + p, ri_g[p], m_g[p])
        return carry

    lax.fori_loop(0, PAIRS_PER_W // GROUP, group_body, 0)

    for h in range(HALVES):
        out_cp(i0 + PAIRS_PER_W - 1, h).wait()


@jax.jit
def _rel_pos_sc(ri, mask_i32, table):
    mesh = plsc.VectorSubcoreMesh(core_axis_name="c", subcore_axis_name="s")
    f = pl.kernel(
        _sc_body,
        out_type=jax.ShapeDtypeStruct((ROWS, D_), jnp.float32),
        mesh=mesh,
        scratch_types=[
            pltpu.VMEM((L_ + LANES,), jnp.int32),
            pltpu.VMEM((L_ + LANES,), jnp.int32),
            pltpu.VMEM((2 * BINS_ + 2, D_), jnp.float32),
            pltpu.VMEM((HALVES, HL, D_), jnp.float32),
            pltpu.SemaphoreType.DMA((HALVES,)),
        ],
    )
    return f(ri, mask_i32, table)


def kernel(residue_index, mask, table):
    out = _rel_pos_sc(residue_index, mask.astype(jnp.int32), table)
    return out.reshape(B_, L_, L_, D_)
